# baseline (speedup 1.0000x reference)
import functools
import math

import jax
import jax.numpy as jnp
from jax import lax
from jax.experimental import pallas as pl
from jax.experimental.pallas import tpu as pltpu
from jax.experimental.pallas import tpu_sc as plsc

D_H = 2048
E = 64
SCALE = 1.0 / math.sqrt(2048.0)

NC = 2
NS = 16
NW = NC * NS
RPW = D_H // NW
CH = 16
NCH = RPW // CH
L = 16
NV = D_H // L


_GDN = lax.GatherDimensionNumbers(
    offset_dims=(), collapsed_slice_dims=(0,), start_index_map=(0,))


def _lane_shuffle(x, idx):
    return lax.gather(x, idx[:, None], dimension_numbers=_GDN,
                      slice_sizes=(1,),
                      mode=lax.GatherScatterMode.PROMISE_IN_BOUNDS)


def _lane_allreduce(x, perms):
    for p in perms:
        x = x + _lane_shuffle(x, p)
    return x


def _sc_body(wq_hbm, wk_hbm, ctx_hbm, out_hbm, wq_v, wk_v, ctx_v, t_v):
    c = lax.axis_index("c")
    s = lax.axis_index("s")
    wid = s * NC + c
    base = wid * RPW

    pltpu.sync_copy(ctx_hbm, ctx_v)

    def _zero(j, carry):
        t_v[pl.ds(j * L, L)] = jnp.zeros((L,), jnp.float32)
        return carry

    lax.fori_loop(0, NV, _zero, 0)

    def _chunk(ci, carry):
        row0 = base + ci * CH
        pltpu.sync_copy(wq_hbm.at[pl.ds(row0, CH)], wq_v)
        pltpu.sync_copy(wk_hbm.at[pl.ds(row0, CH)], wk_v)

        def _qstep(j, accs):
            cv = ctx_v[pl.ds(j * L, L)]
            return tuple(accs[g] + wq_v[g, pl.ds(j * L, L)] * cv
                         for g in range(CH))

        accs = lax.fori_loop(
            0, NV, _qstep,
            tuple(jnp.zeros((L,), jnp.float32) for _ in range(CH)))
        lanes = lax.iota(jnp.int32, L)
        perms = [jnp.bitwise_xor(lanes, jnp.int32(st)) for st in (1, 2, 4, 8)]
        qs = [_lane_allreduce(a, perms) for a in accs]

        def _tstep(j, carry2):
            tv = t_v[pl.ds(j * L, L)]
            for g in range(CH):
                tv = tv + qs[g] * wk_v[g, pl.ds(j * L, L)]
            t_v[pl.ds(j * L, L)] = tv
            return carry2

        lax.fori_loop(0, NV, _tstep, 0)
        return carry

    lax.fori_loop(0, NCH, _chunk, 0)
    pltpu.sync_copy(t_v, out_hbm.at[wid])


def _fin_body(part_ref, keys_ref, out_ref):
    t = jnp.sum(part_ref[...], axis=0, keepdims=True)
    sc = jax.lax.dot_general(
        t, keys_ref[...], (((1,), (1,)), ((), ())),
        preferred_element_type=jnp.float32) * SCALE
    m = jnp.max(sc, axis=-1, keepdims=True)
    ex = jnp.exp(sc - m)
    out_ref[...] = ex / jnp.sum(ex, axis=-1, keepdims=True)


def kernel(expert_outputs, context, keys, Wq, Wk):
    del expert_outputs

    sc_fn = functools.partial(
        pl.kernel,
        mesh=plsc.VectorSubcoreMesh(core_axis_name="c", subcore_axis_name="s"),
        out_type=jax.ShapeDtypeStruct((NW, D_H), jnp.float32),
        scratch_types=[
            pltpu.VMEM((CH, D_H), jnp.float32),
            pltpu.VMEM((CH, D_H), jnp.float32),
            pltpu.VMEM((D_H,), jnp.float32),
            pltpu.VMEM((D_H,), jnp.float32),
        ],
    )(_sc_body)
    partials = sc_fn(Wq, Wk, context)

    gate = pl.pallas_call(
        _fin_body,
        in_specs=[
            pl.BlockSpec((NW, D_H), lambda: (0, 0)),
            pl.BlockSpec((E, D_H), lambda: (0, 0)),
        ],
        out_specs=pl.BlockSpec((1, E), lambda: (0, 0)),
        out_shape=jax.ShapeDtypeStruct((1, E), jnp.float32),
    )(partials, keys)
    return gate.reshape(E)


# speedup vs baseline: 1.5810x; 1.5810x over previous
import functools
import math

import jax
import jax.numpy as jnp
from jax import lax
from jax.experimental import pallas as pl
from jax.experimental.pallas import tpu as pltpu
from jax.experimental.pallas import tpu_sc as plsc

D_H = 2048
E = 64
SCALE = 1.0 / math.sqrt(2048.0)

SC_ROWS = 512
TC_ROWS = D_H - SC_ROWS
NB = 3
R = TC_ROWS // NB

NC = 2
NS = 16
NW = NC * NS
RPW = SC_ROWS // NW
CH = 16
NCH = RPW // CH
L = 16
NV = D_H // L

_GDN = lax.GatherDimensionNumbers(
    offset_dims=(), collapsed_slice_dims=(0,), start_index_map=(0,))


def _lane_shuffle(x, idx):
    return lax.gather(x, idx[:, None], dimension_numbers=_GDN,
                      slice_sizes=(1,),
                      mode=lax.GatherScatterMode.PROMISE_IN_BOUNDS)


def _lane_allreduce(x, perms):
    for p in perms:
        x = x + _lane_shuffle(x, p)
    return x


def _sc_body(wq_hbm, wk_hbm, ctx_hbm, out_hbm, wq_v, wk_v, ctx_v, t_v):
    c = lax.axis_index("c")
    s = lax.axis_index("s")
    wid = s * NC + c
    base = TC_ROWS + wid * RPW

    pltpu.sync_copy(ctx_hbm, ctx_v)

    def _zero(j, carry):
        t_v[pl.ds(j * L, L)] = jnp.zeros((L,), jnp.float32)
        return carry

    lax.fori_loop(0, NV, _zero, 0)

    def _chunk(ci, carry):
        row0 = base + ci * CH
        pltpu.sync_copy(wq_hbm.at[pl.ds(row0, CH)], wq_v)
        pltpu.sync_copy(wk_hbm.at[pl.ds(row0, CH)], wk_v)

        def _qstep(j, accs):
            cv = ctx_v[pl.ds(j * L, L)]
            return tuple(accs[g] + wq_v[g, pl.ds(j * L, L)] * cv
                         for g in range(CH))

        accs = lax.fori_loop(
            0, NV, _qstep,
            tuple(jnp.zeros((L,), jnp.float32) for _ in range(CH)))
        lanes = lax.iota(jnp.int32, L)
        perms = [jnp.bitwise_xor(lanes, jnp.int32(st)) for st in (1, 2, 4, 8)]
        qs = [_lane_allreduce(a, perms) for a in accs]

        def _tstep(j, carry2):
            tv = t_v[pl.ds(j * L, L)]
            for g in range(CH):
                tv = tv + qs[g] * wk_v[g, pl.ds(j * L, L)]
            t_v[pl.ds(j * L, L)] = tv
            return carry2

        lax.fori_loop(0, NV, _tstep, 0)
        return carry

    lax.fori_loop(0, NCH, _chunk, 0)
    pltpu.sync_copy(t_v, out_hbm.at[wid])


def _tc_body(ctx_ref, wq_ref, wk_ref, out_ref, t_ref):
    i = pl.program_id(0)

    @pl.when(i == 0)
    def _init():
        t_ref[...] = jnp.zeros_like(t_ref)

    q_blk = jax.lax.dot_general(
        ctx_ref[...], wq_ref[...], (((1,), (1,)), ((), ())),
        preferred_element_type=jnp.float32)
    t_ref[...] += jax.lax.dot_general(
        q_blk, wk_ref[...], (((1,), (0,)), ((), ())),
        preferred_element_type=jnp.float32)

    @pl.when(i == NB - 1)
    def _fin():
        out_ref[...] = t_ref[...]


def _fin_body(t_tc_ref, part_ref, keys_ref, out_ref):
    t = t_tc_ref[...] + jnp.sum(part_ref[...], axis=0, keepdims=True)
    sc = jax.lax.dot_general(
        t, keys_ref[...], (((1,), (1,)), ((), ())),
        preferred_element_type=jnp.float32) * SCALE
    m = jnp.max(sc, axis=-1, keepdims=True)
    ex = jnp.exp(sc - m)
    out_ref[...] = ex / jnp.sum(ex, axis=-1, keepdims=True)


def kernel(expert_outputs, context, keys, Wq, Wk):
    del expert_outputs
    ctx2 = context.reshape(1, D_H)

    sc_fn = functools.partial(
        pl.kernel,
        mesh=plsc.VectorSubcoreMesh(core_axis_name="c", subcore_axis_name="s"),
        out_type=jax.ShapeDtypeStruct((NW, D_H), jnp.float32),
        scratch_types=[
            pltpu.VMEM((CH, D_H), jnp.float32),
            pltpu.VMEM((CH, D_H), jnp.float32),
            pltpu.VMEM((D_H,), jnp.float32),
            pltpu.VMEM((D_H,), jnp.float32),
        ],
    )(_sc_body)
    partials = sc_fn(Wq, Wk, context)

    t_tc = pl.pallas_call(
        _tc_body,
        grid=(NB,),
        in_specs=[
            pl.BlockSpec((1, D_H), lambda i: (0, 0)),
            pl.BlockSpec((R, D_H), lambda i: (i, 0)),
            pl.BlockSpec((R, D_H), lambda i: (i, 0)),
        ],
        out_specs=pl.BlockSpec((1, D_H), lambda i: (0, 0)),
        out_shape=jax.ShapeDtypeStruct((1, D_H), jnp.float32),
        scratch_shapes=[pltpu.VMEM((1, D_H), jnp.float32)],
        compiler_params=pltpu.CompilerParams(
            dimension_semantics=("arbitrary",),
        ),
    )(ctx2, Wq, Wk)

    gate = pl.pallas_call(
        _fin_body,
        in_specs=[
            pl.BlockSpec((1, D_H), lambda: (0, 0)),
            pl.BlockSpec((NW, D_H), lambda: (0, 0)),
            pl.BlockSpec((E, D_H), lambda: (0, 0)),
        ],
        out_specs=pl.BlockSpec((1, E), lambda: (0, 0)),
        out_shape=jax.ShapeDtypeStruct((1, E), jnp.float32),
    )(t_tc, partials, keys)
    return gate.reshape(E)


# speedup vs baseline: 4.1124x; 2.6012x over previous
import math

import jax
import jax.numpy as jnp
from jax.experimental import pallas as pl
from jax.experimental.pallas import tpu as pltpu

D_H = 2048
E = 64
NB = 4
R = D_H // (2 * NB)
SCALE = 1.0 / math.sqrt(2048.0)


def _chain(ctx, wq, wk):
    q = jax.lax.dot_general(
        ctx, wq, (((1,), (1,)), ((), ())), preferred_element_type=jnp.float32)
    return jax.lax.dot_general(
        q, wk, (((1,), (0,)), ((), ())), preferred_element_type=jnp.float32)


def _body(ctx_ref, wqa_ref, wqb_ref, wka_ref, wkb_ref, keys_ref, out_ref,
          t_ref):
    i = pl.program_id(0)

    @pl.when(i == 0)
    def _init():
        t_ref[...] = jnp.zeros_like(t_ref)

    ctx = ctx_ref[...]
    t_ref[...] += (_chain(ctx, wqa_ref[...], wka_ref[...]) +
                   _chain(ctx, wqb_ref[...], wkb_ref[...]))

    @pl.when(i == NB - 1)
    def _fin():
        s = jax.lax.dot_general(
            t_ref[...], keys_ref[...], (((1,), (1,)), ((), ())),
            preferred_element_type=jnp.float32) * SCALE
        m = jnp.max(s, axis=-1, keepdims=True)
        ex = jnp.exp(s - m)
        out_ref[...] = ex / jnp.sum(ex, axis=-1, keepdims=True)


def kernel(expert_outputs, context, keys, Wq, Wk):
    del expert_outputs
    ctx2 = context.reshape(1, D_H)
    gate = pl.pallas_call(
        _body,
        grid=(NB,),
        in_specs=[
            pl.BlockSpec((1, D_H), lambda i: (0, 0)),
            pl.BlockSpec((R, D_H), lambda i: (i, 0)),
            pl.BlockSpec((R, D_H), lambda i: (i + NB, 0)),
            pl.BlockSpec((R, D_H), lambda i: (i, 0)),
            pl.BlockSpec((R, D_H), lambda i: (i + NB, 0)),
            pl.BlockSpec((E, D_H), lambda i: (0, 0)),
        ],
        out_specs=pl.BlockSpec((1, E), lambda i: (0, 0)),
        out_shape=jax.ShapeDtypeStruct((1, E), jnp.float32),
        scratch_shapes=[pltpu.VMEM((1, D_H), jnp.float32)],
        compiler_params=pltpu.CompilerParams(
            dimension_semantics=("arbitrary",),
        ),
    )(ctx2, Wq, Wq, Wk, Wk, keys)
    return gate.reshape(E)
